# baseline (speedup 1.0000x reference)
import functools

import jax
import jax.numpy as jnp
from jax import lax
from jax.experimental import pallas as pl
from jax.experimental.pallas import tpu as pltpu
from jax.experimental.pallas import tpu_sc as plsc

K = 1024
CB = 4
D = 64
N = 16384
RB = 512
NBLK = N // RB
BETA = 0.25

NC = 2
NS = 16
NW = NC * NS
TOT = CB * N
ROWS_W = TOT // NW
CH = 512
NCH = ROWS_W // CH
GCH = 128
NG = CH // GCH


def _tc_body(x_ref, emb_ref, idx_ref, loss_ref):
    g = pl.program_id(0)

    @pl.when(g == 0)
    def _():
        loss_ref[:, :] = jnp.zeros((1, 1), jnp.float32)

    acc = jnp.zeros((), jnp.float32)
    for i in range(CB):
        a = x_ref[:, i:i + D]
        w = emb_ref[i]
        c = lax.dot_general(a, w, (((1,), (1,)), ((), ())),
                            preferred_element_type=jnp.float32)
        xsq = jnp.sum(a * a, axis=1, keepdims=True)
        wsq = jnp.sum(w * w, axis=1)
        dist = (xsq + wsq[None, :]) - 2.0 * c
        md = jnp.min(dist, axis=1)
        iota = lax.broadcasted_iota(jnp.int32, (RB, K), 1)
        arg = jnp.min(jnp.where(dist == md[:, None], iota, K), axis=1)
        idx_ref[i, :] = arg + i * K
        acc = acc + jnp.sum(md)

    loss_ref[:, :] = loss_ref[:, :] + (acc * ((1.0 + BETA) / (N * D))).reshape(1, 1)


def _tc_call(xt, emb):
    return pl.pallas_call(
        _tc_body,
        grid=(NBLK,),
        in_specs=[
            pl.BlockSpec((RB, CB * D), lambda g: (g, 0)),
            pl.BlockSpec((CB, K, D), lambda g: (0, 0, 0)),
        ],
        out_specs=[
            pl.BlockSpec((CB, RB), lambda g: (0, g)),
            pl.BlockSpec((1, 1), lambda g: (0, 0)),
        ],
        out_shape=[
            jax.ShapeDtypeStruct((CB, N), jnp.int32),
            jax.ShapeDtypeStruct((1, 1), jnp.float32),
        ],
    )(xt, emb)


def _sc_gather_body(table_hbm, idx_hbm, out_hbm, idx_v, rows_v, sem):
    wid = lax.axis_index("s") * NC + lax.axis_index("c")
    base = wid * ROWS_W
    pltpu.sync_copy(idx_hbm.at[pl.ds(base, ROWS_W)], idx_v)

    def chunk(ci, carry):
        cps = []
        for gj in range(NG):
            cp = pltpu.async_copy(
                table_hbm.at[idx_v.at[pl.ds(ci * CH + gj * GCH, GCH)]],
                rows_v.at[pl.ds(gj * GCH, GCH)], sem)
            cps.append(cp)
        for cp in cps:
            cp.wait()
        pltpu.sync_copy(rows_v, out_hbm.at[pl.ds(base + ci * CH, CH)])
        return carry

    lax.fori_loop(0, NCH, chunk, 0)


@functools.cache
def _sc_gather():
    return pl.kernel(
        _sc_gather_body,
        out_type=jax.ShapeDtypeStruct((TOT, D), jnp.float32),
        mesh=plsc.VectorSubcoreMesh(core_axis_name="c", subcore_axis_name="s"),
        scratch_types=[
            pltpu.VMEM((ROWS_W,), jnp.int32),
            pltpu.VMEM((CH, D), jnp.float32),
            pltpu.SemaphoreType.DMA,
        ],
        compiler_params=pltpu.CompilerParams(use_tc_tiling_on_sc=False),
    )


def kernel(latents, emb):
    B, C, H, W = latents.shape
    xt = jnp.transpose(latents, (0, 2, 3, 1)).reshape(N, C)
    idx, loss2d = _tc_call(xt, emb)
    table = emb.reshape(CB * K, D)
    q = _sc_gather()(table, idx.reshape(-1))
    quant = q.reshape(CB, B, H, W, D).transpose(1, 0, 4, 2, 3).reshape(B, C, H, W)
    return quant, loss2d[0, 0]


# speedup vs baseline: 1.1922x; 1.1922x over previous
import functools

import jax
import jax.numpy as jnp
from jax import lax
from jax.experimental import pallas as pl
from jax.experimental.pallas import tpu as pltpu
from jax.experimental.pallas import tpu_sc as plsc

K = 1024
CB = 4
D = 64
N = 16384
RB = 512
NBLK = N // RB
KC = 256
BETA = 0.25

NC = 2
NS = 16
NW = NC * NS
TOT = CB * N
ROWS_W = TOT // NW
CH = 512
NCH = ROWS_W // CH
GCH = 128
NG = CH // GCH


def _tc_body(x_ref, emb_ref, idx_ref, loss_ref, wsq_ref):
    g = pl.program_id(0)

    @pl.when(g == 0)
    def _():
        loss_ref[:, :] = jnp.zeros((1, 1), jnp.float32)
        for i in range(CB):
            w = emb_ref[i]
            wsq_ref[i:i + 1, :] = jnp.sum(w * w, axis=1)[None, :]

    acc = jnp.zeros((), jnp.float32)
    iota = lax.broadcasted_iota(jnp.int32, (RB, K), 1)
    for i in range(CB):
        a = x_ref[:, i:i + D]
        w2 = emb_ref[i] + emb_ref[i]
        c2 = lax.dot_general(a, w2, (((1,), (1,)), ((), ())),
                             preferred_element_type=jnp.float32)
        xsq = jnp.sum(a * a, axis=1, keepdims=True)
        dist = (xsq + wsq_ref[i:i + 1, :]) - c2
        md = jnp.min(dist, axis=1, keepdims=True)
        arg = jnp.min(jnp.where(dist == md, iota, K), axis=1, keepdims=True)
        idx_ref[:, i:i + 1] = arg + i * K
        acc = acc + jnp.sum(md)

    loss_ref[:, :] = loss_ref[:, :] + (acc * ((1.0 + BETA) / (N * D))).reshape(1, 1)


def _tc_call(xt, emb):
    return pl.pallas_call(
        _tc_body,
        grid=(NBLK,),
        in_specs=[
            pl.BlockSpec((RB, CB * D), lambda g: (g, 0)),
            pl.BlockSpec((CB, K, D), lambda g: (0, 0, 0)),
        ],
        out_specs=[
            pl.BlockSpec((RB, CB), lambda g: (g, 0)),
            pl.BlockSpec((1, 1), lambda g: (0, 0)),
        ],
        out_shape=[
            jax.ShapeDtypeStruct((N, CB), jnp.int32),
            jax.ShapeDtypeStruct((1, 1), jnp.float32),
        ],
        scratch_shapes=[pltpu.VMEM((CB, K), jnp.float32)],
    )(xt, emb)


def _sc_gather_body(table_hbm, idx_hbm, out_hbm, idx_v, rows_v, sem):
    wid = lax.axis_index("s") * NC + lax.axis_index("c")
    base = wid * ROWS_W
    pltpu.sync_copy(idx_hbm.at[pl.ds(base, ROWS_W)], idx_v)

    def chunk(ci, carry):
        cps = []
        for gj in range(NG):
            cp = pltpu.async_copy(
                table_hbm.at[idx_v.at[pl.ds(ci * CH + gj * GCH, GCH)]],
                rows_v.at[pl.ds(gj * GCH, GCH)], sem)
            cps.append(cp)
        for cp in cps:
            cp.wait()
        pltpu.sync_copy(rows_v, out_hbm.at[pl.ds(base + ci * CH, CH)])
        return carry

    lax.fori_loop(0, NCH, chunk, 0)


@functools.cache
def _sc_gather():
    return pl.kernel(
        _sc_gather_body,
        out_type=jax.ShapeDtypeStruct((TOT, D), jnp.float32),
        mesh=plsc.VectorSubcoreMesh(core_axis_name="c", subcore_axis_name="s"),
        scratch_types=[
            pltpu.VMEM((ROWS_W,), jnp.int32),
            pltpu.VMEM((CH, D), jnp.float32),
            pltpu.SemaphoreType.DMA,
        ],
        compiler_params=pltpu.CompilerParams(use_tc_tiling_on_sc=False),
    )


def kernel(latents, emb):
    B, C, H, W = latents.shape
    xt = jnp.transpose(latents, (0, 2, 3, 1)).reshape(N, C)
    idx, loss2d = _tc_call(xt, emb)
    table = emb.reshape(CB * K, D)
    q = _sc_gather()(table, idx.reshape(-1))
    quant = q.reshape(B, H, W, CB, D).transpose(0, 3, 4, 1, 2).reshape(B, C, H, W)
    return quant, loss2d[0, 0]
